# baseline (speedup 1.0000x reference)
import functools

import jax
import jax.numpy as jnp
from jax import lax
from jax.experimental import pallas as pl
from jax.experimental.pallas import tpu as pltpu
from jax.experimental.pallas import tpu_sc as plsc

_NC = 2
_NS = 16
_NW = _NC * _NS
_L = 16
_C = 128
_BB = 512


def _edge_body(E, hst_ref, hen_ref, dp_ref, a_ref, b_ref, c_ref,
               be1_ref, we2_ref, be2_ref, winf_ref, binf_ref, w_ref):
    gid = pl.program_id(0)

    @pl.when(gid * _BB < E)
    def _():
        dp = dp_ref[...].reshape(_BB // _C, _C)
        sel = (lax.broadcasted_iota(jnp.int32, (_BB, _BB // _C), 0) // _C
               == lax.broadcasted_iota(jnp.int32, (_BB, _BB // _C), 1)
               ).astype(jnp.float32)
        dbc = jnp.dot(sel, dp, preferred_element_type=jnp.float32)
        lmask = (lax.broadcasted_iota(jnp.int32, (_BB, _C), 0) % _C
                 == lax.broadcasted_iota(jnp.int32, (_BB, _C), 1))
        dist = jnp.sum(jnp.where(lmask, dbc, 0.0), axis=1, keepdims=True)

        z1 = (jnp.dot(hst_ref[...].astype(jnp.bfloat16), a_ref[...],
                      preferred_element_type=jnp.float32)
              + jnp.dot(hen_ref[...].astype(jnp.bfloat16), b_ref[...],
                        preferred_element_type=jnp.float32)
              + dist * c_ref[...] + be1_ref[...])
        s1 = z1 * jax.nn.sigmoid(z1)
        z2 = jnp.dot(s1.astype(jnp.bfloat16), we2_ref[...],
                     preferred_element_type=jnp.float32) + be2_ref[...]
        m = z2 * jax.nn.sigmoid(z2)
        e = jax.nn.sigmoid(jnp.sum(m * winf_ref[...], axis=1, keepdims=True)
                           + binf_ref[...])
        w_ref[...] = e * m

    @pl.when(gid * _BB >= E)
    def _():
        w_ref[...] = jnp.zeros_like(w_ref)


def _node_body(h_ref, p_ref, u_ref, v_ref, bh1_ref, wh2_ref, bh2_ref, out_ref):
    h = h_ref[...]
    m = p_ref[0] + p_ref[1]
    z = (jnp.dot(h.astype(jnp.bfloat16), u_ref[...],
                 preferred_element_type=jnp.float32)
         + jnp.dot(m.astype(jnp.bfloat16), v_ref[...],
                   preferred_element_type=jnp.float32)
         + bh1_ref[...])
    s = z * jax.nn.sigmoid(z)
    out_ref[...] = h + jnp.dot(s.astype(jnp.bfloat16), wh2_ref[...],
                               preferred_element_type=jnp.float32) + bh2_ref[...]


@functools.cache
def _make_edge_call(E, E_pad, D):
    grid = E_pad // _BB
    full = lambda i: (0, 0)
    return pl.pallas_call(
        functools.partial(_edge_body, E),
        grid=(grid,),
        in_specs=[
            pl.BlockSpec((_BB, D), lambda i: (i, 0)),
            pl.BlockSpec((_BB, D), lambda i: (i, 0)),
            pl.BlockSpec((_BB // _C, 1, _C), lambda i: (i, 0, 0)),
            pl.BlockSpec((D, D), full),
            pl.BlockSpec((D, D), full),
            pl.BlockSpec((1, D), full),
            pl.BlockSpec((1, D), full),
            pl.BlockSpec((D, D), full),
            pl.BlockSpec((1, D), full),
            pl.BlockSpec((1, D), full),
            pl.BlockSpec((1, 1), full),
        ],
        out_specs=pl.BlockSpec((_BB, D), lambda i: (i, 0)),
        out_shape=jax.ShapeDtypeStruct((E_pad, D), jnp.float32),
    )


@functools.cache
def _make_node_call(N, D):
    bn = 1000
    grid = N // bn
    full = lambda i: (0, 0)
    return pl.pallas_call(
        _node_body,
        grid=(grid,),
        in_specs=[
            pl.BlockSpec((bn, D), lambda i: (i, 0)),
            pl.BlockSpec((_NC, bn, D), lambda i: (0, i, 0)),
            pl.BlockSpec((D, D), full),
            pl.BlockSpec((D, D), full),
            pl.BlockSpec((1, D), full),
            pl.BlockSpec((D, D), full),
            pl.BlockSpec((1, D), full),
        ],
        out_specs=pl.BlockSpec((bn, D), lambda i: (i, 0)),
        out_shape=jax.ShapeDtypeStruct((N, D), jnp.float32),
    )


def _sc_dist(st16, en16, xf_v):
    a = st16 * 4
    b = en16 * 4
    dx = plsc.load_gather(xf_v, [a]) - plsc.load_gather(xf_v, [b])
    dy = plsc.load_gather(xf_v, [a + 1]) - plsc.load_gather(xf_v, [b + 1])
    dz = plsc.load_gather(xf_v, [a + 2]) - plsc.load_gather(xf_v, [b + 2])
    d2 = dx * dx + dy * dy + dz * dz
    i = jnp.full((_L,), 0x5F3759DF, jnp.int32) - lax.shift_right_logical(
        plsc.bitcast(d2, jnp.int32), 1)
    r = plsc.bitcast(i, jnp.float32)
    r = r * (1.5 - 0.5 * d2 * r * r)
    r = r * (1.5 - 0.5 * d2 * r * r)
    r = r * (1.5 - 0.5 * d2 * r * r)
    return jnp.where(d2 > 0.0, d2 * r, 0.0)


@functools.cache
def _make_dist_call(N, E_pad):
    K = E_pad // (_NW * _C)
    n_chunks = E_pad // _C
    mesh = plsc.VectorSubcoreMesh(core_axis_name="c", subcore_axis_name="s",
                                  num_cores=_NC, num_subcores=_NS)
    f32 = jnp.float32

    @functools.partial(
        pl.kernel,
        out_type=jax.ShapeDtypeStruct((n_chunks, 1, _C), f32),
        mesh=mesh,
        scratch_types=[
            pltpu.VMEM((K, 1, _C), jnp.int32),
            pltpu.VMEM((K, 1, _C), jnp.int32),
            pltpu.VMEM((4 * N,), f32),
            pltpu.VMEM((1, _C), f32),
        ],
        compiler_params=pltpu.CompilerParams(needs_layout_passes=False),
    )
    def dist(xf_hbm, st_hbm, en_hbm, dp_out, st_v, en_v, xf_v, db):
        wid = lax.axis_index("c") * _NS + lax.axis_index("s")
        cbase = wid * K
        pltpu.sync_copy(st_hbm.at[pl.ds(cbase, K)], st_v)
        pltpu.sync_copy(en_hbm.at[pl.ds(cbase, K)], en_v)
        pltpu.sync_copy(xf_hbm, xf_v)

        def body(j, carry):
            for g in range(_C // _L):
                sl = pl.ds(g * _L, _L)
                db[0, sl] = _sc_dist(st_v[j, 0, sl], en_v[j, 0, sl], xf_v)
            pltpu.sync_copy(db, dp_out.at[cbase + j])
            return carry

        lax.fori_loop(0, K, body, 0)

    return dist


_C2 = 64
_G = 8


@functools.cache
def _make_gather_call(N_pad, D, E_pad):
    K2 = E_pad // (_NW * _C2)
    NG = K2 // _G
    rows_pt = N_pad // _NS
    mesh = plsc.VectorSubcoreMesh(core_axis_name="c", subcore_axis_name="s",
                                  num_cores=_NC, num_subcores=_NS)
    f32 = jnp.float32

    @functools.partial(
        pl.kernel,
        out_type=(
            jax.ShapeDtypeStruct((E_pad, D), f32),
            jax.ShapeDtypeStruct((E_pad, D), f32),
        ),
        mesh=mesh,
        scratch_types=[
            pltpu.VMEM_SHARED((N_pad, D), f32),
            pltpu.VMEM((_G, 1, _C2), jnp.int32),
            pltpu.VMEM((_G, 1, _C2), jnp.int32),
            pltpu.VMEM((2, _C2, D), f32),
            pltpu.VMEM((2, _C2, D), f32),
            pltpu.SemaphoreType.DMA,
            pltpu.SemaphoreType.DMA,
            pltpu.SemaphoreType.DMA,
            pltpu.SemaphoreType.DMA,
        ],
    )
    def gather(h_hbm, st_hbm, en_hbm, hst_out, hen_out,
               h_sh, st_r, en_r, hbs, hbe, s0a, s0b, s1a, s1b):
        sid = lax.axis_index("s")
        wid = lax.axis_index("c") * _NS + sid
        cbase = wid * K2
        r0 = sid * rows_pt
        pltpu.sync_copy(h_hbm.at[pl.ds(r0, rows_pt)],
                        h_sh.at[pl.ds(r0, rows_pt)])
        plsc.subcore_barrier()
        ss = (s0a, s1a)
        se = (s0b, s1b)

        def start(r, b):
            pltpu.async_copy(h_sh.at[st_r.at[r, 0]], hbs.at[b], ss[b])
            pltpu.async_copy(h_sh.at[en_r.at[r, 0]], hbe.at[b], se[b])

        def drain(b):
            pltpu.make_async_copy(h_sh.at[st_r.at[0, 0]], hbs.at[b], ss[b]).wait()
            pltpu.make_async_copy(h_sh.at[en_r.at[0, 0]], hbe.at[b], se[b]).wait()

        def body(g, carry):
            pltpu.sync_copy(st_hbm.at[pl.ds(cbase + g * _G, _G)], st_r)
            pltpu.sync_copy(en_hbm.at[pl.ds(cbase + g * _G, _G)], en_r)
            start(0, 0)
            for b in range(_G):
                if b + 1 < _G:
                    start(b + 1, (b + 1) % 2)
                drain(b % 2)
                row0 = (cbase + g * _G + b) * _C2
                pltpu.sync_copy(hbs.at[b % 2], hst_out.at[pl.ds(row0, _C2)])
                pltpu.sync_copy(hbe.at[b % 2], hen_out.at[pl.ds(row0, _C2)])
            return carry

        lax.fori_loop(0, NG, body, 0)

    return gather


@functools.cache
def _make_scatter_call(N_pad, D, E_pad):
    K = E_pad // (_NW * _C)
    rows_pt = N_pad // _NS
    mesh = plsc.VectorSubcoreMesh(core_axis_name="c", subcore_axis_name="s",
                                  num_cores=_NC, num_subcores=_NS)
    f32 = jnp.float32

    @functools.partial(
        pl.kernel,
        out_type=jax.ShapeDtypeStruct((_NC, N_pad, D), f32),
        mesh=mesh,
        scratch_types=[
            pltpu.VMEM_SHARED((N_pad, D), f32),
            pltpu.VMEM((K, 1, _C), jnp.int32),
            pltpu.VMEM((2, _C, D), f32),
            pltpu.SemaphoreType.DMA,
            pltpu.SemaphoreType.DMA,
        ],
    )
    def scatter(w_hbm, st_hbm, zeros_hbm, out_hbm, shared, st_v, wbuf, r0s, r1s):
        cid = lax.axis_index("c")
        sid = lax.axis_index("s")
        r0 = sid * rows_pt
        pltpu.sync_copy(zeros_hbm.at[pl.ds(r0, rows_pt)],
                        shared.at[pl.ds(r0, rows_pt)])
        plsc.subcore_barrier()
        cbase = (cid * _NS + sid) * K
        pltpu.sync_copy(st_hbm.at[pl.ds(cbase, K)], st_v)
        rs = (r0s, r1s)

        def startw(j, b):
            pltpu.async_copy(w_hbm.at[pl.ds((cbase + j) * _C, _C)],
                             wbuf.at[b], rs[b])

        def drainw(b):
            pltpu.make_async_copy(w_hbm.at[pl.ds(0, _C)], wbuf.at[b],
                                  rs[b]).wait()

        startw(0, 0)

        def body(i, carry):
            for b in (0, 1):
                j = 2 * i + b
                startw(jnp.minimum(j + 1, K - 1), 1 - b)
                drainw(b)
                pltpu.sync_copy(wbuf.at[b], shared.at[st_v.at[j, 0]], add=True)
            return carry

        lax.fori_loop(0, K // 2, body, 0)
        drainw(0)
        plsc.subcore_barrier()
        pltpu.sync_copy(shared.at[pl.ds(r0, rows_pt)],
                        out_hbm.at[cid].at[pl.ds(r0, rows_pt)])

    return scatter


def kernel(x, h, edges, We1, be1, We2, be2, Winf, binf, Wh1, bh1, Wh2, bh2):
    N, D = h.shape
    E = edges.shape[0]
    cpw = _C * 8
    per_w = -(-E // (_NW * cpw)) * cpw
    E_pad = per_w * _NW

    st = edges[:, 0]
    en = edges[:, 1]
    pad = E_pad - E
    st3 = jnp.pad(st, (0, pad)).reshape(-1, 1, _C)
    en3 = jnp.pad(en, (0, pad)).reshape(-1, 1, _C)
    xf = jnp.pad(x, ((0, 0), (0, 4 - x.shape[1]))).reshape(-1)

    n_pad = -(-N // (_NS * 8)) * _NS * 8
    hp = jnp.pad(h, ((0, n_pad - N), (0, 0)))
    st64 = st3.reshape(-1, 1, _C2)
    en64 = en3.reshape(-1, 1, _C2)
    dp = _make_dist_call(N, E_pad)(xf, st3, en3)
    hst, hen = _make_gather_call(n_pad, D, E_pad)(hp, st64, en64)

    bf16 = jnp.bfloat16
    w = _make_edge_call(E, E_pad, D)(
        hst, hen, dp,
        We1[:D].astype(bf16), We1[D:2 * D].astype(bf16),
        We1[2 * D].reshape(1, D),
        be1.reshape(1, D), We2.astype(bf16), be2.reshape(1, D),
        Winf.reshape(1, D), binf.reshape(1, 1))

    p = _make_scatter_call(n_pad, D, E_pad)(
        w, st3, jnp.zeros((n_pad, D), jnp.float32))

    return _make_node_call(N, D)(
        h, p, Wh1[:D].astype(bf16), Wh1[D:].astype(bf16),
        bh1.reshape(1, D), Wh2.astype(bf16), bh2.reshape(1, D))


# speedup vs baseline: 1.1379x; 1.1379x over previous
import functools

import jax
import jax.numpy as jnp
from jax import lax
from jax.experimental import pallas as pl
from jax.experimental.pallas import tpu as pltpu
from jax.experimental.pallas import tpu_sc as plsc

_NC = 2
_NS = 16
_NW = _NC * _NS
_L = 16
_C = 128
_BB = 2560


def _edge_body(E, hst_ref, hen_ref, dp_ref, a_ref, b_ref, c_ref,
               be1_ref, we2_ref, be2_ref, winf_ref, binf_ref, w_ref):
    gid = pl.program_id(0)

    @pl.when(gid * _BB < E)
    def _():
        dp = dp_ref[...].reshape(_BB // _C, _C)
        sel = (lax.broadcasted_iota(jnp.int32, (_BB, _BB // _C), 0) // _C
               == lax.broadcasted_iota(jnp.int32, (_BB, _BB // _C), 1)
               ).astype(jnp.float32)
        dbc = jnp.dot(sel, dp, preferred_element_type=jnp.float32)
        lmask = (lax.broadcasted_iota(jnp.int32, (_BB, _C), 0) % _C
                 == lax.broadcasted_iota(jnp.int32, (_BB, _C), 1))
        dist = jnp.sum(jnp.where(lmask, dbc, 0.0), axis=1, keepdims=True)

        z1 = (jnp.dot(hst_ref[...].astype(jnp.bfloat16), a_ref[...],
                      preferred_element_type=jnp.float32)
              + jnp.dot(hen_ref[...].astype(jnp.bfloat16), b_ref[...],
                        preferred_element_type=jnp.float32)
              + dist * c_ref[...] + be1_ref[...])
        s1 = z1 * jax.nn.sigmoid(z1)
        z2 = jnp.dot(s1.astype(jnp.bfloat16), we2_ref[...],
                     preferred_element_type=jnp.float32) + be2_ref[...]
        m = z2 * jax.nn.sigmoid(z2)
        e = jax.nn.sigmoid(jnp.sum(m * winf_ref[...], axis=1, keepdims=True)
                           + binf_ref[...])
        w_ref[...] = e * m

    @pl.when(gid * _BB >= E)
    def _():
        w_ref[...] = jnp.zeros_like(w_ref)


def _node_body(h_ref, p_ref, u_ref, v_ref, bh1_ref, wh2_ref, bh2_ref, out_ref):
    h = h_ref[...]
    m = p_ref[0] + p_ref[1]
    z = (jnp.dot(h.astype(jnp.bfloat16), u_ref[...],
                 preferred_element_type=jnp.float32)
         + jnp.dot(m.astype(jnp.bfloat16), v_ref[...],
                   preferred_element_type=jnp.float32)
         + bh1_ref[...])
    s = z * jax.nn.sigmoid(z)
    out_ref[...] = h + jnp.dot(s.astype(jnp.bfloat16), wh2_ref[...],
                               preferred_element_type=jnp.float32) + bh2_ref[...]


@functools.cache
def _make_edge_call(E, E_pad, D):
    grid = E_pad // _BB
    full = lambda i: (0, 0)
    return pl.pallas_call(
        functools.partial(_edge_body, E),
        grid=(grid,),
        in_specs=[
            pl.BlockSpec((_BB, D), lambda i: (i, 0)),
            pl.BlockSpec((_BB, D), lambda i: (i, 0)),
            pl.BlockSpec((_BB // _C, 1, _C), lambda i: (i, 0, 0)),
            pl.BlockSpec((D, D), full),
            pl.BlockSpec((D, D), full),
            pl.BlockSpec((1, D), full),
            pl.BlockSpec((1, D), full),
            pl.BlockSpec((D, D), full),
            pl.BlockSpec((1, D), full),
            pl.BlockSpec((1, D), full),
            pl.BlockSpec((1, 1), full),
        ],
        out_specs=pl.BlockSpec((_BB, D), lambda i: (i, 0)),
        out_shape=jax.ShapeDtypeStruct((E_pad, D), jnp.float32),
    )


@functools.cache
def _make_node_call(N, D):
    bn = 1000
    grid = N // bn
    full = lambda i: (0, 0)
    return pl.pallas_call(
        _node_body,
        grid=(grid,),
        in_specs=[
            pl.BlockSpec((bn, D), lambda i: (i, 0)),
            pl.BlockSpec((_NC, bn, D), lambda i: (0, i, 0)),
            pl.BlockSpec((D, D), full),
            pl.BlockSpec((D, D), full),
            pl.BlockSpec((1, D), full),
            pl.BlockSpec((D, D), full),
            pl.BlockSpec((1, D), full),
        ],
        out_specs=pl.BlockSpec((bn, D), lambda i: (i, 0)),
        out_shape=jax.ShapeDtypeStruct((N, D), jnp.float32),
    )


def _sc_dist(st16, en16, xf_v):
    a = st16 * 4
    b = en16 * 4
    dx = plsc.load_gather(xf_v, [a]) - plsc.load_gather(xf_v, [b])
    dy = plsc.load_gather(xf_v, [a + 1]) - plsc.load_gather(xf_v, [b + 1])
    dz = plsc.load_gather(xf_v, [a + 2]) - plsc.load_gather(xf_v, [b + 2])
    d2 = dx * dx + dy * dy + dz * dz
    i = jnp.full((_L,), 0x5F3759DF, jnp.int32) - lax.shift_right_logical(
        plsc.bitcast(d2, jnp.int32), 1)
    r = plsc.bitcast(i, jnp.float32)
    r = r * (1.5 - 0.5 * d2 * r * r)
    r = r * (1.5 - 0.5 * d2 * r * r)
    r = r * (1.5 - 0.5 * d2 * r * r)
    return jnp.where(d2 > 0.0, d2 * r, 0.0)


@functools.cache
def _make_dist_call(N, E_pad):
    K = E_pad // (_NW * _C)
    n_chunks = E_pad // _C
    mesh = plsc.VectorSubcoreMesh(core_axis_name="c", subcore_axis_name="s",
                                  num_cores=_NC, num_subcores=_NS)
    f32 = jnp.float32

    @functools.partial(
        pl.kernel,
        out_type=jax.ShapeDtypeStruct((n_chunks, 1, _C), f32),
        mesh=mesh,
        scratch_types=[
            pltpu.VMEM((K, 1, _C), jnp.int32),
            pltpu.VMEM((K, 1, _C), jnp.int32),
            pltpu.VMEM((4 * N,), f32),
            pltpu.VMEM((1, _C), f32),
        ],
        compiler_params=pltpu.CompilerParams(needs_layout_passes=False),
    )
    def dist(xf_hbm, st_hbm, en_hbm, dp_out, st_v, en_v, xf_v, db):
        wid = lax.axis_index("c") * _NS + lax.axis_index("s")
        cbase = wid * K
        pltpu.sync_copy(st_hbm.at[pl.ds(cbase, K)], st_v)
        pltpu.sync_copy(en_hbm.at[pl.ds(cbase, K)], en_v)
        pltpu.sync_copy(xf_hbm, xf_v)

        def body(j, carry):
            for g in range(_C // _L):
                sl = pl.ds(g * _L, _L)
                db[0, sl] = _sc_dist(st_v[j, 0, sl], en_v[j, 0, sl], xf_v)
            pltpu.sync_copy(db, dp_out.at[cbase + j])
            return carry

        lax.fori_loop(0, K, body, 0)

    return dist


_C2 = 64
_G = 8


@functools.cache
def _make_gather_call(N_pad, D, E_pad):
    K2 = E_pad // (_NW * _C2)
    NG = K2 // _G
    rows_pt = N_pad // _NS
    mesh = plsc.VectorSubcoreMesh(core_axis_name="c", subcore_axis_name="s",
                                  num_cores=_NC, num_subcores=_NS)
    f32 = jnp.float32

    @functools.partial(
        pl.kernel,
        out_type=(
            jax.ShapeDtypeStruct((E_pad, D), f32),
            jax.ShapeDtypeStruct((E_pad, D), f32),
        ),
        mesh=mesh,
        scratch_types=[
            pltpu.VMEM_SHARED((N_pad, D), f32),
            pltpu.VMEM((_G, 1, _C2), jnp.int32),
            pltpu.VMEM((_G, 1, _C2), jnp.int32),
            pltpu.VMEM((2, _C2, D), f32),
            pltpu.VMEM((2, _C2, D), f32),
            pltpu.SemaphoreType.DMA,
            pltpu.SemaphoreType.DMA,
            pltpu.SemaphoreType.DMA,
            pltpu.SemaphoreType.DMA,
        ],
    )
    def gather(h_hbm, st_hbm, en_hbm, hst_out, hen_out,
               h_sh, st_r, en_r, hbs, hbe, s0a, s0b, s1a, s1b):
        sid = lax.axis_index("s")
        wid = lax.axis_index("c") * _NS + sid
        cbase = wid * K2
        r0 = sid * rows_pt
        pltpu.sync_copy(h_hbm.at[pl.ds(r0, rows_pt)],
                        h_sh.at[pl.ds(r0, rows_pt)])
        plsc.subcore_barrier()
        ss = (s0a, s1a)
        se = (s0b, s1b)

        def start(r, b):
            pltpu.async_copy(h_sh.at[st_r.at[r, 0]], hbs.at[b], ss[b])
            pltpu.async_copy(h_sh.at[en_r.at[r, 0]], hbe.at[b], se[b])

        def drain(b):
            pltpu.make_async_copy(h_sh.at[st_r.at[0, 0]], hbs.at[b], ss[b]).wait()
            pltpu.make_async_copy(h_sh.at[en_r.at[0, 0]], hbe.at[b], se[b]).wait()

        def body(g, carry):
            pltpu.sync_copy(st_hbm.at[pl.ds(cbase + g * _G, _G)], st_r)
            pltpu.sync_copy(en_hbm.at[pl.ds(cbase + g * _G, _G)], en_r)
            start(0, 0)
            for b in range(_G):
                if b + 1 < _G:
                    start(b + 1, (b + 1) % 2)
                drain(b % 2)
                row0 = (cbase + g * _G + b) * _C2
                pltpu.sync_copy(hbs.at[b % 2], hst_out.at[pl.ds(row0, _C2)])
                pltpu.sync_copy(hbe.at[b % 2], hen_out.at[pl.ds(row0, _C2)])
            return carry

        lax.fori_loop(0, NG, body, 0)

    return gather


@functools.cache
def _make_scatter_call(N_pad, D, E_pad):
    K = E_pad // (_NW * _C)
    rows_pt = N_pad // _NS
    mesh = plsc.VectorSubcoreMesh(core_axis_name="c", subcore_axis_name="s",
                                  num_cores=_NC, num_subcores=_NS)
    f32 = jnp.float32

    @functools.partial(
        pl.kernel,
        out_type=jax.ShapeDtypeStruct((_NC, N_pad, D), f32),
        mesh=mesh,
        scratch_types=[
            pltpu.VMEM_SHARED((N_pad, D), f32),
            pltpu.VMEM((K, 1, _C), jnp.int32),
            pltpu.VMEM((2, _C, D), f32),
            pltpu.SemaphoreType.DMA,
            pltpu.SemaphoreType.DMA,
        ],
    )
    def scatter(w_hbm, st_hbm, zeros_hbm, out_hbm, shared, st_v, wbuf, r0s, r1s):
        cid = lax.axis_index("c")
        sid = lax.axis_index("s")
        r0 = sid * rows_pt
        pltpu.sync_copy(zeros_hbm.at[pl.ds(r0, rows_pt)],
                        shared.at[pl.ds(r0, rows_pt)])
        plsc.subcore_barrier()
        cbase = (cid * _NS + sid) * K
        pltpu.sync_copy(st_hbm.at[pl.ds(cbase, K)], st_v)
        rs = (r0s, r1s)

        def startw(j, b):
            pltpu.async_copy(w_hbm.at[pl.ds((cbase + j) * _C, _C)],
                             wbuf.at[b], rs[b])

        def drainw(b):
            pltpu.make_async_copy(w_hbm.at[pl.ds(0, _C)], wbuf.at[b],
                                  rs[b]).wait()

        startw(0, 0)

        def body(i, carry):
            for b in (0, 1):
                j = 2 * i + b
                startw(jnp.minimum(j + 1, K - 1), 1 - b)
                drainw(b)
                pltpu.sync_copy(wbuf.at[b], shared.at[st_v.at[j, 0]], add=True)
            return carry

        lax.fori_loop(0, K // 2, body, 0)
        drainw(0)
        plsc.subcore_barrier()
        pltpu.sync_copy(shared.at[pl.ds(r0, rows_pt)],
                        out_hbm.at[cid].at[pl.ds(r0, rows_pt)])

    return scatter


def kernel(x, h, edges, We1, be1, We2, be2, Winf, binf, Wh1, bh1, Wh2, bh2):
    N, D = h.shape
    E = edges.shape[0]
    cpw = _C * 8
    per_w = -(-E // (_NW * cpw)) * cpw
    E_pad = per_w * _NW

    st = edges[:, 0]
    en = edges[:, 1]
    pad = E_pad - E
    st3 = jnp.pad(st, (0, pad)).reshape(-1, 1, _C)
    en3 = jnp.pad(en, (0, pad)).reshape(-1, 1, _C)
    xf = jnp.pad(x, ((0, 0), (0, 4 - x.shape[1]))).reshape(-1)

    n_pad = -(-N // (_NS * 8)) * _NS * 8
    hp = jnp.pad(h, ((0, n_pad - N), (0, 0)))
    st64 = st3.reshape(-1, 1, _C2)
    en64 = en3.reshape(-1, 1, _C2)
    dp = _make_dist_call(N, E_pad)(xf, st3, en3)
    hst, hen = _make_gather_call(n_pad, D, E_pad)(hp, st64, en64)

    bf16 = jnp.bfloat16
    w = _make_edge_call(E, E_pad, D)(
        hst, hen, dp,
        We1[:D].astype(bf16), We1[D:2 * D].astype(bf16),
        We1[2 * D].reshape(1, D),
        be1.reshape(1, D), We2.astype(bf16), be2.reshape(1, D),
        Winf.reshape(1, D), binf.reshape(1, 1))

    p = _make_scatter_call(n_pad, D, E_pad)(
        w, st3, jnp.zeros((n_pad, D), jnp.float32))

    return _make_node_call(N, D)(
        h, p, Wh1[:D].astype(bf16), Wh1[D:].astype(bf16),
        bh1.reshape(1, D), Wh2.astype(bf16), bh2.reshape(1, D))
